# baseline (speedup 1.0000x reference)
import jax
import jax.numpy as jnp
from jax.experimental import pallas as pl

_B, _N, _D, _H = 4, 2048, 128, 256
_NS, _K = 6, 10
_RB = 256

_PREC = None


def _dot(a, b, dims):
    return jax.lax.dot_general(
        a, b, (dims, ((), ())), precision=_PREC,
        preferred_element_type=jnp.float32)


def _body(conf_ref, confT_ref, nat_r_ref, nat_c_ref,
          w1_ref, b1r_ref, b1c_ref, w2_ref, b2r_ref, b2c_ref,
          w3_ref, b3_ref, ws1_ref, bs1r_ref, ws2_ref, bs2r_ref,
          e_ref, basin_ref, depth_ref, width_ref, sidx_ref,
          logits_ref, metrics_ref):
    N, RB, K = _N, _RB, _K
    conf = conf_ref[0]
    confT = confT_ref[0]
    nat_r = nat_r_ref[0]
    nat_c = nat_c_ref[0]
    w1 = w1_ref[...]
    w2 = w2_ref[...]
    w3 = w3_ref[...]

    h1 = jax.nn.relu(_dot(conf, w1, ((1,), (0,))) + b1r_ref[...])
    h2 = jax.nn.relu(_dot(h1, w2, ((1,), (0,))) + b2r_ref[...])
    e_col = _dot(h2, w3, ((1,), (0,))) + b3_ref[...]
    e_ref[0] = e_col

    g1 = jax.nn.relu(_dot(w1, confT, ((0,), (0,))) + b1c_ref[...])
    g2 = jax.nn.relu(_dot(w2, g1, ((0,), (0,))) + b2c_ref[...])
    e_row = _dot(w3, g2, ((0,), (0,))) + b3_ref[...]

    hs = jax.nn.relu(_dot(conf, ws1_ref[...], ((1,), (0,))) + bs1r_ref[...])
    logits = _dot(hs, ws2_ref[...], ((1,), (0,))) + bs2r_ref[...]
    logits_ref[0] = logits
    mx = jnp.max(logits, axis=1, keepdims=True)
    i6 = jax.lax.broadcasted_iota(jnp.int32, (N, _NS), 1)
    sidx_ref[0] = jnp.min(jnp.where(logits == mx, i6, _NS),
                          axis=1, keepdims=True)

    x2_row = jnp.sum(confT * confT, axis=0, keepdims=True)
    ndifT = confT - nat_c
    nd_row = jnp.sqrt(jnp.sum(ndifT * ndifT, axis=0, keepdims=True))
    ndif = conf - nat_r
    nd_col = jnp.sqrt(jnp.sum(ndif * ndif, axis=1, keepdims=True))

    BIG = jnp.float32(3.0e38)
    ones_col = jnp.ones((N, 1), jnp.float32)

    def blk(rb, se_row):
        off = rb * RB
        cb = conf_ref[0, pl.ds(off, RB), :]
        x2b = jnp.sum(cb * cb, axis=1, keepdims=True)
        cc = _dot(cb, confT, ((1,), (0,)))
        d2 = x2b + x2_row - 2.0 * cc
        dist = jnp.sqrt(jnp.maximum(d2, 1e-12))
        eb = e_ref[0, pl.ds(off, RB), :]
        iota = jax.lax.broadcasted_iota(jnp.int32, (RB, N), 1)
        gi = jax.lax.broadcasted_iota(jnp.int32, (RB, N), 0) + off

        work = dist
        kth = jnp.zeros((RB, 1), jnp.float32)
        for k in range(K):
            if k == K - 1:
                kth = jnp.min(work, axis=1, keepdims=True)
            idx = jnp.argmin(work, axis=1, keepdims=True)
            work = jnp.where(iota == idx, BIG, work)
        nbr_min = jnp.min(jnp.where(work == BIG, e_row, BIG),
                          axis=1, keepdims=True)

        basin_ref[0, pl.ds(off, RB), :] = (eb <= nbr_min).astype(jnp.int32)
        maskb = dist < kth
        maskf = jnp.where(maskb, 1.0, 0.0)
        depth_ref[0, pl.ds(off, RB), :] = (
            jnp.max(jnp.where(maskb, e_row, -jnp.inf), axis=1, keepdims=True)
            - eb)
        width_ref[0, pl.ds(off, RB), :] = (
            jnp.sum(dist * maskf, axis=1, keepdims=True)
            / jnp.sum(maskf, axis=1, keepdims=True))
        cbn = cb - nat_r
        ndb = jnp.sqrt(jnp.sum(cbn * cbn, axis=1, keepdims=True))
        gi = jax.lax.broadcasted_iota(jnp.int32, (RB, N), 0) + off
        cmp = (iota != gi) & ((nd_row < ndb)
                              | ((nd_row == ndb) & (iota < gi)))
        rankb = jnp.sum(cmp.astype(jnp.int32), axis=1, keepdims=True)
        se_row = se_row + jnp.sum(jnp.where(rankb == iota, eb, 0.0),
                                  axis=0, keepdims=True)
        return se_row

    se_row = jax.lax.fori_loop(0, N // RB, blk,
                               jnp.zeros((1, N), jnp.float32))

    sum_e = jnp.sum(e_col, axis=0, keepdims=True)
    max_e = jnp.max(e_col, axis=0, keepdims=True)
    mnd = jnp.min(nd_col, axis=0, keepdims=True)
    iota_c = jax.lax.broadcasted_iota(jnp.int32, (N, 1), 0)
    nidx = jnp.min(jnp.where(nd_col == mnd, iota_c, N),
                   axis=0, keepdims=True)
    native_e = jnp.sum(jnp.where(iota_c == nidx, e_col, 0.0),
                       axis=0, keepdims=True)
    ce = e_col - sum_e / N
    rug = jnp.sqrt(jnp.sum(ce * ce, axis=0, keepdims=True) / (N - 1))
    an = nd_col - jnp.sum(nd_col, axis=0, keepdims=True) / N
    corr = (jnp.sum(an * ce, axis=0, keepdims=True)
            / jnp.sqrt(jnp.sum(an * an, axis=0, keepdims=True)
                       * jnp.sum(ce * ce, axis=0, keepdims=True)))
    inc = (se_row[:, 1:] > se_row[:, :-1]).astype(jnp.float32)
    fr = jnp.sum(inc, axis=1, keepdims=True) / (N - 1)
    metrics_ref[0, :, 0:1] = max_e - native_e
    metrics_ref[0, :, 1:2] = rug
    metrics_ref[0, :, 2:3] = corr
    metrics_ref[0, :, 3:4] = fr
    metrics_ref[0, :, 4:5] = native_e


def kernel(conformations, native_state, W1, b1, W2, b2, W3, b3,
           Ws1, bs1, Ws2, bs2):
    B, N, D = conformations.shape
    H = W1.shape[1]
    NS = Ws2.shape[1]
    confT = jnp.swapaxes(conformations, 1, 2)
    nat_r = native_state[:, None, :]
    nat_c = native_state[:, :, None]
    b1r, b1c = b1[None, :], b1[:, None]
    b2r, b2c = b2[None, :], b2[:, None]
    b3s = b3[None, :]
    bs1r = bs1[None, :]
    bs2r = bs2[None, :]

    def im3(b):
        return (b, 0, 0)

    def im2(b):
        return (0, 0)

    f32 = jnp.float32
    outs = pl.pallas_call(
        _body,
        grid=(B,),
        in_specs=[
            pl.BlockSpec((1, N, D), im3),
            pl.BlockSpec((1, D, N), im3),
            pl.BlockSpec((1, 1, D), im3),
            pl.BlockSpec((1, D, 1), im3),
            pl.BlockSpec((D, H), im2),
            pl.BlockSpec((1, H), im2),
            pl.BlockSpec((H, 1), im2),
            pl.BlockSpec((H, H // 2), im2),
            pl.BlockSpec((1, H // 2), im2),
            pl.BlockSpec((H // 2, 1), im2),
            pl.BlockSpec((H // 2, 1), im2),
            pl.BlockSpec((1, 1), im2),
            pl.BlockSpec((D, H), im2),
            pl.BlockSpec((1, H), im2),
            pl.BlockSpec((H, NS), im2),
            pl.BlockSpec((1, NS), im2),
        ],
        out_specs=[
            pl.BlockSpec((1, N, 1), im3),
            pl.BlockSpec((1, N, 1), im3),
            pl.BlockSpec((1, N, 1), im3),
            pl.BlockSpec((1, N, 1), im3),
            pl.BlockSpec((1, N, 1), im3),
            pl.BlockSpec((1, N, NS), im3),
            pl.BlockSpec((1, 1, 5), im3),
        ],
        out_shape=[
            jax.ShapeDtypeStruct((B, N, 1), f32),
            jax.ShapeDtypeStruct((B, N, 1), jnp.int32),
            jax.ShapeDtypeStruct((B, N, 1), f32),
            jax.ShapeDtypeStruct((B, N, 1), f32),
            jax.ShapeDtypeStruct((B, N, 1), jnp.int32),
            jax.ShapeDtypeStruct((B, N, NS), f32),
            jax.ShapeDtypeStruct((B, 1, 5), f32),
        ],
    )(conformations, confT, nat_r, nat_c, W1, b1r, b1c, W2, b2r, b2c,
      W3, b3s, Ws1, bs1r, Ws2, bs2r)
    e3, basin3, depth3, width3, sidx3, logits, metrics3 = outs
    return (e3[..., 0], basin3[..., 0].astype(bool), depth3[..., 0],
            width3[..., 0], sidx3[..., 0], logits, metrics3[:, 0, :])


# speedup vs baseline: 1.2978x; 1.2978x over previous
import jax
import jax.numpy as jnp
from jax.experimental import pallas as pl

_B, _N, _D, _H = 4, 2048, 128, 256
_NS, _K = 6, 10
_RB = 256

_PREC = None


def _dot(a, b, dims):
    return jax.lax.dot_general(
        a, b, (dims, ((), ())), precision=_PREC,
        preferred_element_type=jnp.float32)


def _body(conf_ref, confT_ref, nat_r_ref, nat_c_ref,
          w1_ref, b1r_ref, b1c_ref, w2_ref, b2r_ref, b2c_ref,
          w3_ref, b3_ref, ws1_ref, bs1r_ref, ws2_ref, bs2r_ref,
          e_ref, basin_ref, depth_ref, width_ref, sidx_ref,
          logits_ref, metrics_ref):
    N, RB, K = _N, _RB, _K
    conf = conf_ref[0]
    confT = confT_ref[0]
    nat_r = nat_r_ref[0]
    nat_c = nat_c_ref[0]
    w1 = w1_ref[...]
    w2 = w2_ref[...]
    w3 = w3_ref[...]

    h1 = jax.nn.relu(_dot(conf, w1, ((1,), (0,))) + b1r_ref[...])
    h2 = jax.nn.relu(_dot(h1, w2, ((1,), (0,))) + b2r_ref[...])
    e_col = _dot(h2, w3, ((1,), (0,))) + b3_ref[...]
    e_ref[0] = e_col

    g1 = jax.nn.relu(_dot(w1, confT, ((0,), (0,))) + b1c_ref[...])
    g2 = jax.nn.relu(_dot(w2, g1, ((0,), (0,))) + b2c_ref[...])
    e_row = _dot(w3, g2, ((0,), (0,))) + b3_ref[...]

    hs = jax.nn.relu(_dot(conf, ws1_ref[...], ((1,), (0,))) + bs1r_ref[...])
    logits = _dot(hs, ws2_ref[...], ((1,), (0,))) + bs2r_ref[...]
    logits_ref[0] = logits
    mx = jnp.max(logits, axis=1, keepdims=True)
    i6 = jax.lax.broadcasted_iota(jnp.int32, (N, _NS), 1)
    sidx_ref[0] = jnp.min(jnp.where(logits == mx, i6, _NS),
                          axis=1, keepdims=True)

    x2_row = jnp.sum(confT * confT, axis=0, keepdims=True)
    ndifT = confT - nat_c
    nd_row = jnp.sqrt(jnp.sum(ndifT * ndifT, axis=0, keepdims=True))
    ndif = conf - nat_r
    nd_col = jnp.sqrt(jnp.sum(ndif * ndif, axis=1, keepdims=True))

    BIG = jnp.float32(3.0e38)
    BIG2 = jnp.float32(2.0e38)

    def blk(rb, se_row):
        off = rb * RB
        cb = conf_ref[0, pl.ds(off, RB), :]
        x2b = jnp.sum(cb * cb, axis=1, keepdims=True)
        cc = _dot(cb, confT, ((1,), (0,)))
        d2 = x2b + x2_row - 2.0 * cc
        dist = jnp.sqrt(jnp.maximum(d2, 1e-12))
        eb = e_ref[0, pl.ds(off, RB), :]
        iota = jax.lax.broadcasted_iota(jnp.int32, (RB, N), 1)
        gi = jax.lax.broadcasted_iota(jnp.int32, (RB, N), 0) + off

        work = jnp.where(iota == gi, BIG2, dist)
        kth = jnp.zeros((RB, 1), jnp.float32)
        for k in range(K - 1):
            m = jnp.min(work, axis=1, keepdims=True)
            if k == K - 2:
                kth = m
            work = jnp.where(work == m, BIG, work)
        marks = work == BIG
        tot = jnp.sum(jnp.where(marks, 1.0, 0.0), axis=1, keepdims=True)
        nbr_min = jnp.min(jnp.where(marks, e_row, BIG),
                          axis=1, keepdims=True)

        def exact_topk(_):
            w = dist
            kv = jnp.zeros((RB, 1), jnp.float32)
            i0 = jnp.zeros((RB, 1), jnp.int32)
            for k in range(K):
                if k == K - 1:
                    kv = jnp.min(w, axis=1, keepdims=True)
                ix = jnp.argmin(w, axis=1, keepdims=True)
                if k == 0:
                    i0 = ix
                w = jnp.where(iota == ix, BIG, w)
            nm = jnp.min(
                jnp.where((w == BIG) & (iota != i0), e_row, BIG),
                axis=1, keepdims=True)
            return kv, nm

        kth, nbr_min = jax.lax.cond(
            jnp.any(tot != jnp.float32(K - 1)),
            exact_topk, lambda _: (kth, nbr_min), None)

        basin_ref[0, pl.ds(off, RB), :] = (eb <= nbr_min).astype(jnp.int32)
        maskb = dist < kth
        maskf = jnp.where(maskb, 1.0, 0.0)
        depth_ref[0, pl.ds(off, RB), :] = (
            jnp.max(jnp.where(maskb, e_row, -jnp.inf), axis=1, keepdims=True)
            - eb)
        width_ref[0, pl.ds(off, RB), :] = (
            jnp.sum(dist * maskf, axis=1, keepdims=True)
            / jnp.sum(maskf, axis=1, keepdims=True))
        cbn = cb - nat_r
        ndb = jnp.sqrt(jnp.sum(cbn * cbn, axis=1, keepdims=True))
        cmp = (iota != gi) & (nd_row < ndb)
        rankb = jnp.sum(cmp.astype(jnp.int32), axis=1, keepdims=True)
        se_row = se_row + jnp.sum(jnp.where(rankb == iota, eb, 0.0),
                                  axis=0, keepdims=True)
        return se_row

    se_row = jax.lax.fori_loop(0, N // RB, blk,
                               jnp.zeros((1, N), jnp.float32))

    sum_e = jnp.sum(e_col, axis=0, keepdims=True)
    max_e = jnp.max(e_col, axis=0, keepdims=True)
    mnd = jnp.min(nd_col, axis=0, keepdims=True)
    iota_c = jax.lax.broadcasted_iota(jnp.int32, (N, 1), 0)
    nidx = jnp.min(jnp.where(nd_col == mnd, iota_c, N),
                   axis=0, keepdims=True)
    native_e = jnp.sum(jnp.where(iota_c == nidx, e_col, 0.0),
                       axis=0, keepdims=True)
    ce = e_col - sum_e / N
    rug = jnp.sqrt(jnp.sum(ce * ce, axis=0, keepdims=True) / (N - 1))
    an = nd_col - jnp.sum(nd_col, axis=0, keepdims=True) / N
    corr = (jnp.sum(an * ce, axis=0, keepdims=True)
            / jnp.sqrt(jnp.sum(an * an, axis=0, keepdims=True)
                       * jnp.sum(ce * ce, axis=0, keepdims=True)))
    inc = (se_row[:, 1:] > se_row[:, :-1]).astype(jnp.float32)
    fr = jnp.sum(inc, axis=1, keepdims=True) / (N - 1)
    metrics_ref[0, :, 0:1] = max_e - native_e
    metrics_ref[0, :, 1:2] = rug
    metrics_ref[0, :, 2:3] = corr
    metrics_ref[0, :, 3:4] = fr
    metrics_ref[0, :, 4:5] = native_e


def kernel(conformations, native_state, W1, b1, W2, b2, W3, b3,
           Ws1, bs1, Ws2, bs2):
    B, N, D = conformations.shape
    H = W1.shape[1]
    NS = Ws2.shape[1]
    confT = jnp.swapaxes(conformations, 1, 2)
    nat_r = native_state[:, None, :]
    nat_c = native_state[:, :, None]
    b1r, b1c = b1[None, :], b1[:, None]
    b2r, b2c = b2[None, :], b2[:, None]
    b3s = b3[None, :]
    bs1r = bs1[None, :]
    bs2r = bs2[None, :]

    def im3(b):
        return (b, 0, 0)

    def im2(b):
        return (0, 0)

    f32 = jnp.float32
    outs = pl.pallas_call(
        _body,
        grid=(B,),
        in_specs=[
            pl.BlockSpec((1, N, D), im3),
            pl.BlockSpec((1, D, N), im3),
            pl.BlockSpec((1, 1, D), im3),
            pl.BlockSpec((1, D, 1), im3),
            pl.BlockSpec((D, H), im2),
            pl.BlockSpec((1, H), im2),
            pl.BlockSpec((H, 1), im2),
            pl.BlockSpec((H, H // 2), im2),
            pl.BlockSpec((1, H // 2), im2),
            pl.BlockSpec((H // 2, 1), im2),
            pl.BlockSpec((H // 2, 1), im2),
            pl.BlockSpec((1, 1), im2),
            pl.BlockSpec((D, H), im2),
            pl.BlockSpec((1, H), im2),
            pl.BlockSpec((H, NS), im2),
            pl.BlockSpec((1, NS), im2),
        ],
        out_specs=[
            pl.BlockSpec((1, N, 1), im3),
            pl.BlockSpec((1, N, 1), im3),
            pl.BlockSpec((1, N, 1), im3),
            pl.BlockSpec((1, N, 1), im3),
            pl.BlockSpec((1, N, 1), im3),
            pl.BlockSpec((1, N, NS), im3),
            pl.BlockSpec((1, 1, 5), im3),
        ],
        out_shape=[
            jax.ShapeDtypeStruct((B, N, 1), f32),
            jax.ShapeDtypeStruct((B, N, 1), jnp.int32),
            jax.ShapeDtypeStruct((B, N, 1), f32),
            jax.ShapeDtypeStruct((B, N, 1), f32),
            jax.ShapeDtypeStruct((B, N, 1), jnp.int32),
            jax.ShapeDtypeStruct((B, N, NS), f32),
            jax.ShapeDtypeStruct((B, 1, 5), f32),
        ],
    )(conformations, confT, nat_r, nat_c, W1, b1r, b1c, W2, b2r, b2c,
      W3, b3s, Ws1, bs1r, Ws2, bs2r)
    e3, basin3, depth3, width3, sidx3, logits, metrics3 = outs
    return (e3[..., 0], basin3[..., 0].astype(bool), depth3[..., 0],
            width3[..., 0], sidx3[..., 0], logits, metrics3[:, 0, :])


# speedup vs baseline: 1.3429x; 1.0348x over previous
import jax
import jax.numpy as jnp
from jax.experimental import pallas as pl

_B, _N, _D, _H = 4, 2048, 128, 256
_NS, _K = 6, 10
_RB = 512

_PREC = None


def _dot(a, b, dims):
    return jax.lax.dot_general(
        a, b, (dims, ((), ())), precision=_PREC,
        preferred_element_type=jnp.float32)


def _body(conf_ref, confT_ref, nat_r_ref, nat_c_ref,
          w1_ref, b1r_ref, b1c_ref, w2_ref, b2r_ref, b2c_ref,
          w3_ref, b3_ref, ws1_ref, bs1r_ref, ws2_ref, bs2r_ref,
          e_ref, basin_ref, depth_ref, width_ref, sidx_ref,
          logits_ref, metrics_ref):
    N, RB, K = _N, _RB, _K
    conf = conf_ref[0]
    confT = confT_ref[0]
    nat_r = nat_r_ref[0]
    nat_c = nat_c_ref[0]
    w1 = w1_ref[...]
    w2 = w2_ref[...]
    w3 = w3_ref[...]

    h1 = jax.nn.relu(_dot(conf, w1, ((1,), (0,))) + b1r_ref[...])
    h2 = jax.nn.relu(_dot(h1, w2, ((1,), (0,))) + b2r_ref[...])
    e_col = _dot(h2, w3, ((1,), (0,))) + b3_ref[...]
    e_ref[0] = e_col

    g1 = jax.nn.relu(_dot(w1, confT, ((0,), (0,))) + b1c_ref[...])
    g2 = jax.nn.relu(_dot(w2, g1, ((0,), (0,))) + b2c_ref[...])
    e_row = _dot(w3, g2, ((0,), (0,))) + b3_ref[...]

    hs = jax.nn.relu(_dot(conf, ws1_ref[...], ((1,), (0,))) + bs1r_ref[...])
    logits = _dot(hs, ws2_ref[...], ((1,), (0,))) + bs2r_ref[...]
    logits_ref[0] = logits
    mx = jnp.max(logits, axis=1, keepdims=True)
    i6 = jax.lax.broadcasted_iota(jnp.int32, (N, _NS), 1)
    sidx_ref[0] = jnp.min(jnp.where(logits == mx, i6, _NS),
                          axis=1, keepdims=True)

    x2_row = jnp.sum(confT * confT, axis=0, keepdims=True)
    ndifT = confT - nat_c
    nd_row = jnp.sqrt(jnp.sum(ndifT * ndifT, axis=0, keepdims=True))
    ndif = conf - nat_r
    nd_col = jnp.sqrt(jnp.sum(ndif * ndif, axis=1, keepdims=True))

    BIG = jnp.float32(3.0e38)
    BIG2 = jnp.float32(2.0e38)

    def blk(rb, se_row):
        off = rb * RB
        cb = conf_ref[0, pl.ds(off, RB), :]
        x2b = jnp.sum(cb * cb, axis=1, keepdims=True)
        cc = _dot(cb, confT, ((1,), (0,)))
        d2 = x2b + x2_row - 2.0 * cc
        dist = jnp.sqrt(jnp.maximum(d2, 1e-12))
        eb = e_ref[0, pl.ds(off, RB), :]
        iota = jax.lax.broadcasted_iota(jnp.int32, (RB, N), 1)
        gi = jax.lax.broadcasted_iota(jnp.int32, (RB, N), 0) + off

        work = jnp.where(iota == gi, BIG2, dist)
        kth = jnp.zeros((RB, 1), jnp.float32)
        for k in range(K - 1):
            m = jnp.min(work, axis=1, keepdims=True)
            if k == K - 2:
                kth = m
            work = jnp.where(work == m, BIG, work)
        marks = work == BIG
        tot = jnp.sum(jnp.where(marks, 1.0, 0.0), axis=1, keepdims=True)
        nbr_min = jnp.min(jnp.where(marks, e_row, BIG),
                          axis=1, keepdims=True)

        def exact_topk(_):
            w = dist
            kv = jnp.zeros((RB, 1), jnp.float32)
            i0 = jnp.zeros((RB, 1), jnp.int32)
            for k in range(K):
                if k == K - 1:
                    kv = jnp.min(w, axis=1, keepdims=True)
                ix = jnp.argmin(w, axis=1, keepdims=True)
                if k == 0:
                    i0 = ix
                w = jnp.where(iota == ix, BIG, w)
            nm = jnp.min(
                jnp.where((w == BIG) & (iota != i0), e_row, BIG),
                axis=1, keepdims=True)
            return kv, nm

        kth, nbr_min = jax.lax.cond(
            jnp.any(tot != jnp.float32(K - 1)),
            exact_topk, lambda _: (kth, nbr_min), None)

        basin_ref[0, pl.ds(off, RB), :] = (eb <= nbr_min).astype(jnp.int32)
        maskb = dist < kth
        maskf = jnp.where(maskb, 1.0, 0.0)
        depth_ref[0, pl.ds(off, RB), :] = (
            jnp.max(jnp.where(maskb, e_row, -jnp.inf), axis=1, keepdims=True)
            - eb)
        width_ref[0, pl.ds(off, RB), :] = (
            jnp.sum(dist * maskf, axis=1, keepdims=True)
            / jnp.sum(maskf, axis=1, keepdims=True))
        cbn = cb - nat_r
        ndb = jnp.sqrt(jnp.sum(cbn * cbn, axis=1, keepdims=True))
        cmp = (iota != gi) & (nd_row < ndb)
        rankb = jnp.sum(cmp.astype(jnp.int32), axis=1, keepdims=True)
        se_row = se_row + jnp.sum(jnp.where(rankb == iota, eb, 0.0),
                                  axis=0, keepdims=True)
        return se_row

    se_row = jax.lax.fori_loop(0, N // RB, blk,
                               jnp.zeros((1, N), jnp.float32))

    sum_e = jnp.sum(e_col, axis=0, keepdims=True)
    max_e = jnp.max(e_col, axis=0, keepdims=True)
    mnd = jnp.min(nd_col, axis=0, keepdims=True)
    iota_c = jax.lax.broadcasted_iota(jnp.int32, (N, 1), 0)
    nidx = jnp.min(jnp.where(nd_col == mnd, iota_c, N),
                   axis=0, keepdims=True)
    native_e = jnp.sum(jnp.where(iota_c == nidx, e_col, 0.0),
                       axis=0, keepdims=True)
    ce = e_col - sum_e / N
    rug = jnp.sqrt(jnp.sum(ce * ce, axis=0, keepdims=True) / (N - 1))
    an = nd_col - jnp.sum(nd_col, axis=0, keepdims=True) / N
    corr = (jnp.sum(an * ce, axis=0, keepdims=True)
            / jnp.sqrt(jnp.sum(an * an, axis=0, keepdims=True)
                       * jnp.sum(ce * ce, axis=0, keepdims=True)))
    inc = (se_row[:, 1:] > se_row[:, :-1]).astype(jnp.float32)
    fr = jnp.sum(inc, axis=1, keepdims=True) / (N - 1)
    metrics_ref[0, :, 0:1] = max_e - native_e
    metrics_ref[0, :, 1:2] = rug
    metrics_ref[0, :, 2:3] = corr
    metrics_ref[0, :, 3:4] = fr
    metrics_ref[0, :, 4:5] = native_e


def kernel(conformations, native_state, W1, b1, W2, b2, W3, b3,
           Ws1, bs1, Ws2, bs2):
    B, N, D = conformations.shape
    H = W1.shape[1]
    NS = Ws2.shape[1]
    confT = jnp.swapaxes(conformations, 1, 2)
    nat_r = native_state[:, None, :]
    nat_c = native_state[:, :, None]
    b1r, b1c = b1[None, :], b1[:, None]
    b2r, b2c = b2[None, :], b2[:, None]
    b3s = b3[None, :]
    bs1r = bs1[None, :]
    bs2r = bs2[None, :]

    def im3(b):
        return (b, 0, 0)

    def im2(b):
        return (0, 0)

    f32 = jnp.float32
    outs = pl.pallas_call(
        _body,
        grid=(B,),
        in_specs=[
            pl.BlockSpec((1, N, D), im3),
            pl.BlockSpec((1, D, N), im3),
            pl.BlockSpec((1, 1, D), im3),
            pl.BlockSpec((1, D, 1), im3),
            pl.BlockSpec((D, H), im2),
            pl.BlockSpec((1, H), im2),
            pl.BlockSpec((H, 1), im2),
            pl.BlockSpec((H, H // 2), im2),
            pl.BlockSpec((1, H // 2), im2),
            pl.BlockSpec((H // 2, 1), im2),
            pl.BlockSpec((H // 2, 1), im2),
            pl.BlockSpec((1, 1), im2),
            pl.BlockSpec((D, H), im2),
            pl.BlockSpec((1, H), im2),
            pl.BlockSpec((H, NS), im2),
            pl.BlockSpec((1, NS), im2),
        ],
        out_specs=[
            pl.BlockSpec((1, N, 1), im3),
            pl.BlockSpec((1, N, 1), im3),
            pl.BlockSpec((1, N, 1), im3),
            pl.BlockSpec((1, N, 1), im3),
            pl.BlockSpec((1, N, 1), im3),
            pl.BlockSpec((1, N, NS), im3),
            pl.BlockSpec((1, 1, 5), im3),
        ],
        out_shape=[
            jax.ShapeDtypeStruct((B, N, 1), f32),
            jax.ShapeDtypeStruct((B, N, 1), jnp.int32),
            jax.ShapeDtypeStruct((B, N, 1), f32),
            jax.ShapeDtypeStruct((B, N, 1), f32),
            jax.ShapeDtypeStruct((B, N, 1), jnp.int32),
            jax.ShapeDtypeStruct((B, N, NS), f32),
            jax.ShapeDtypeStruct((B, 1, 5), f32),
        ],
    )(conformations, confT, nat_r, nat_c, W1, b1r, b1c, W2, b2r, b2c,
      W3, b3s, Ws1, bs1r, Ws2, bs2r)
    e3, basin3, depth3, width3, sidx3, logits, metrics3 = outs
    return (e3[..., 0], basin3[..., 0].astype(bool), depth3[..., 0],
            width3[..., 0], sidx3[..., 0], logits, metrics3[:, 0, :])
